# Initial kernel scaffold; baseline (speedup 1.0000x reference)
import jax
import jax.numpy as jnp
from jax.experimental import pallas as pl


def kernel(hidden_states, Wq, coeffs, W1, b1, W2, b2):
    raise NotImplementedError("write your pallas kernel here")


# speedup vs baseline: 3.4485x; 3.4485x over previous
import jax
import jax.numpy as jnp
from jax.experimental import pallas as pl
from jax.experimental.pallas import tpu as pltpu

_NUM_HEADS = 12
_HEAD_DIM = 64
_POLY_DIM = 256
_MEM_HID = 512
_HIDDEN = 768


def _atlas_body(coeffs_ref, x_ref, wq_ref, w1_ref, b1_ref, w2_ref, b2_ref,
                o_ref):
    c0 = coeffs_ref[0]
    c1 = coeffs_ref[1]
    c2 = coeffs_ref[2]
    c3 = coeffs_ref[3]

    x = x_ref[...]
    q = jnp.dot(x, wq_ref[...], preferred_element_type=jnp.float32)
    xs = jnp.clip(q, -10.0, 10.0)
    f1 = jnp.clip(c1 * xs, -1e6, 1e6)
    xs2 = xs * xs
    f2 = jnp.clip(c2 * xs2, -1e6, 1e6)
    f3 = jnp.clip(c3 * (xs2 * xs), -1e6, 1e6)

    w1 = w1_ref[...]
    b1 = b1_ref[...]
    w2 = w2_ref[...]
    b2 = b2_ref[...]

    t = x.shape[0]
    cblock = jnp.full((t, _HEAD_DIM), c0, dtype=jnp.float32)
    outs = []
    for j in range(_NUM_HEADS):
        sl = slice(j * _HEAD_DIM, (j + 1) * _HEAD_DIM)
        feats = jnp.concatenate([cblock, f1[:, sl], f2[:, sl], f3[:, sl]],
                                axis=-1)
        h = jnp.dot(feats, w1, preferred_element_type=jnp.float32) + b1
        h = jnp.maximum(h, 0.0)
        outs.append(jnp.dot(h, w2, preferred_element_type=jnp.float32) + b2)
    o_ref[...] = jnp.concatenate(outs, axis=-1)


def kernel(hidden_states, Wq, coeffs, W1, b1, W2, b2):
    B, S, H = hidden_states.shape
    x = hidden_states.reshape(B * S, H)
    w2s = W2[:, :_HEAD_DIM]
    b1r = b1.reshape(1, _MEM_HID)
    b2r = b2[:_HEAD_DIM].reshape(1, _HEAD_DIM)

    T = 512
    grid = (B * S // T,)
    out = pl.pallas_call(
        _atlas_body,
        grid=grid,
        in_specs=[
            pl.BlockSpec(memory_space=pltpu.SMEM),
            pl.BlockSpec((T, H), lambda i: (i, 0)),
            pl.BlockSpec((H, H), lambda i: (0, 0)),
            pl.BlockSpec((_POLY_DIM, _MEM_HID), lambda i: (0, 0)),
            pl.BlockSpec((1, _MEM_HID), lambda i: (0, 0)),
            pl.BlockSpec((_MEM_HID, _HEAD_DIM), lambda i: (0, 0)),
            pl.BlockSpec((1, _HEAD_DIM), lambda i: (0, 0)),
        ],
        out_specs=pl.BlockSpec((T, H), lambda i: (i, 0)),
        out_shape=jax.ShapeDtypeStruct((B * S, H), jnp.float32),
        compiler_params=pltpu.CompilerParams(
            dimension_semantics=("parallel",),
        ),
        name="atlas_attention_fused",
    )(coeffs, x, Wq, W1, b1r, w2s, b2r)
    return out.reshape(B, S, _NUM_HEADS * _HEAD_DIM)


# speedup vs baseline: 3.5464x; 1.0284x over previous
import jax
import jax.numpy as jnp
from jax.experimental import pallas as pl
from jax.experimental.pallas import tpu as pltpu

_NUM_HEADS = 12
_HEAD_DIM = 64
_POLY_DIM = 256
_MEM_HID = 512
_HIDDEN = 768


def _atlas_body(coeffs_ref, x_ref, wq_ref, w1_ref, b1_ref, w2_ref, b2_ref,
                o_ref):
    c0 = coeffs_ref[0]
    c1 = coeffs_ref[1]
    c2 = coeffs_ref[2]
    c3 = coeffs_ref[3]

    x = x_ref[...]
    q = jnp.dot(x, wq_ref[...], preferred_element_type=jnp.float32)
    xs = jnp.clip(q, -10.0, 10.0)
    f1 = jnp.clip(c1 * xs, -1e6, 1e6)
    xs2 = xs * xs
    f2 = jnp.clip(c2 * xs2, -1e6, 1e6)
    f3 = jnp.clip(c3 * (xs2 * xs), -1e6, 1e6)

    w1 = w1_ref[...]
    b1 = b1_ref[...]
    w2 = w2_ref[...]
    b2 = b2_ref[...]

    t = x.shape[0]
    cblock = jnp.full((t, _HEAD_DIM), c0, dtype=jnp.float32)
    outs = []
    for j in range(_NUM_HEADS):
        sl = slice(j * _HEAD_DIM, (j + 1) * _HEAD_DIM)
        feats = jnp.concatenate([cblock, f1[:, sl], f2[:, sl], f3[:, sl]],
                                axis=-1)
        h = jnp.dot(feats, w1, preferred_element_type=jnp.float32) + b1
        h = jnp.maximum(h, 0.0)
        outs.append(jnp.dot(h, w2, preferred_element_type=jnp.float32) + b2)
    o_ref[...] = jnp.concatenate(outs, axis=-1)


def kernel(hidden_states, Wq, coeffs, W1, b1, W2, b2):
    B, S, H = hidden_states.shape
    x = hidden_states.reshape(B * S, H)
    w2s = W2[:, :_HEAD_DIM]
    b1r = b1.reshape(1, _MEM_HID)
    b2r = b2[:_HEAD_DIM].reshape(1, _HEAD_DIM)

    T = 1024
    grid = (B * S // T,)
    out = pl.pallas_call(
        _atlas_body,
        grid=grid,
        in_specs=[
            pl.BlockSpec(memory_space=pltpu.SMEM),
            pl.BlockSpec((T, H), lambda i: (i, 0)),
            pl.BlockSpec((H, H), lambda i: (0, 0)),
            pl.BlockSpec((_POLY_DIM, _MEM_HID), lambda i: (0, 0)),
            pl.BlockSpec((1, _MEM_HID), lambda i: (0, 0)),
            pl.BlockSpec((_MEM_HID, _HEAD_DIM), lambda i: (0, 0)),
            pl.BlockSpec((1, _HEAD_DIM), lambda i: (0, 0)),
        ],
        out_specs=pl.BlockSpec((T, H), lambda i: (i, 0)),
        out_shape=jax.ShapeDtypeStruct((B * S, H), jnp.float32),
        compiler_params=pltpu.CompilerParams(
            dimension_semantics=("parallel",),
        ),
        name="atlas_attention_fused",
    )(coeffs, x, Wq, W1, b1r, w2s, b2r)
    return out.reshape(B, S, _NUM_HEADS * _HEAD_DIM)
